# Initial kernel scaffold; baseline (speedup 1.0000x reference)
import jax
import jax.numpy as jnp
from jax.experimental import pallas as pl


def kernel(encodings, categorical):
    raise NotImplementedError("write your pallas kernel here")


# speedup vs baseline: 3.4484x; 3.4484x over previous
import jax
import jax.numpy as jnp
from jax.experimental import pallas as pl

_B = 1024
_D = 64
_C = 16
_K = 25
_MIN_CONF = 0.25
_BIG = 3.0e38


def _overlap_body(enc_ref, cat_ref, ent_ref, ncomp_ref):
    enc = enc_ref[...]
    cat = cat_ref[...]

    sq = jnp.sum(enc * enc, axis=1)
    g = jnp.dot(enc, enc.T, preferred_element_type=jnp.float32)
    d2 = sq[:, None] + sq[None, :] - 2.0 * g
    dist = jnp.sqrt(jnp.maximum(d2, 0.0))

    col = jax.lax.broadcasted_iota(jnp.int32, (_B, _B), 1)

    def extract(_, carry):
        work, _ = carry
        m = jnp.min(work, axis=1)
        jm = jnp.min(jnp.where(work == m[:, None], col, _B), axis=1)
        work = jnp.where(col == jm[:, None], _BIG, work)
        return work, m

    _, thresh = jax.lax.fori_loop(
        0, _K + 1, extract, (dist, jnp.zeros((_B,), jnp.float32))
    )

    mask = (dist < thresh[:, None]).astype(jnp.float32)
    counts = jnp.sum(mask, axis=1)

    cidx = jax.lax.broadcasted_iota(jnp.int32, (_B, _C), 1)
    maxg = jnp.max(cat, axis=1)
    hard = jnp.min(jnp.where(cat == maxg[:, None], cidx, _C), axis=1)
    onehot = (cidx == hard[:, None]).astype(jnp.float32)

    bins = jnp.dot(mask, onehot, preferred_element_type=jnp.float32)
    bins = bins / counts[:, None]
    ent = -jnp.sum(bins * jnp.log(bins + 1e-5), axis=1)
    ent_ref[...] = ent[:, None]

    conf = (maxg >= _MIN_CONF).astype(jnp.float32)
    populated = jnp.sum(onehot * conf[:, None], axis=0)
    ncomp_ref[...] = jnp.sum((populated > 0.0).astype(jnp.float32)).reshape(1, 1)


def kernel(encodings, categorical):
    ent, ncomp = pl.pallas_call(
        _overlap_body,
        out_shape=[
            jax.ShapeDtypeStruct((_B, 1), jnp.float32),
            jax.ShapeDtypeStruct((1, 1), jnp.float32),
        ],
    )(encodings, categorical)
    return encodings, ent.reshape(_B), ncomp.reshape(())


# speedup vs baseline: 5.2023x; 1.5086x over previous
import jax
import jax.numpy as jnp
from jax.experimental import pallas as pl

_B = 1024
_D = 64
_C = 16
_K = 25
_MIN_CONF = 0.25
_BIG = 3.0e38


def _overlap_body(enc_ref, cat_ref, ent_ref, ncomp_ref):
    enc = enc_ref[...]
    cat = cat_ref[...]

    sq = jnp.sum(enc * enc, axis=1)
    g = jnp.dot(enc, enc.T, preferred_element_type=jnp.float32)
    d2 = sq[:, None] + sq[None, :] - 2.0 * g

    m0 = jnp.min(d2, axis=1)
    c0 = jnp.sum((d2 == m0[:, None]).astype(jnp.float32), axis=1)

    def next_distinct(_, carry):
        m, cum, thr = carry
        above = jnp.where(d2 > m[:, None], d2, _BIG)
        mn = jnp.min(above, axis=1)
        cn = jnp.sum((d2 == mn[:, None]).astype(jnp.float32), axis=1)
        thr = jnp.where(cum <= float(_K), mn, thr)
        return mn, cum + cn, thr

    _, _, thresh2 = jax.lax.fori_loop(0, _K, next_distinct, (m0, c0, m0))

    thresh = jnp.sqrt(jnp.maximum(thresh2, 0.0))
    dist = jnp.sqrt(jnp.maximum(d2, 0.0))
    mask = (dist < thresh[:, None]).astype(jnp.float32)
    counts = jnp.sum(mask, axis=1)

    cidx = jax.lax.broadcasted_iota(jnp.int32, (_B, _C), 1)
    maxg = jnp.max(cat, axis=1)
    hard = jnp.min(jnp.where(cat == maxg[:, None], cidx, _C), axis=1)
    onehot = (cidx == hard[:, None]).astype(jnp.float32)

    bins = jnp.dot(mask, onehot, preferred_element_type=jnp.float32)
    bins = bins / counts[:, None]
    ent = -jnp.sum(bins * jnp.log(bins + 1e-5), axis=1)
    ent_ref[...] = ent[:, None]

    conf = (maxg >= _MIN_CONF).astype(jnp.float32)
    populated = jnp.sum(onehot * conf[:, None], axis=0)
    ncomp_ref[...] = jnp.sum((populated > 0.0).astype(jnp.float32)).reshape(1, 1)


def kernel(encodings, categorical):
    ent, ncomp = pl.pallas_call(
        _overlap_body,
        out_shape=[
            jax.ShapeDtypeStruct((_B, 1), jnp.float32),
            jax.ShapeDtypeStruct((1, 1), jnp.float32),
        ],
    )(encodings, categorical)
    return encodings, ent.reshape(_B), ncomp.reshape(())


# speedup vs baseline: 5.6519x; 1.0864x over previous
import jax
import jax.numpy as jnp
from jax.experimental import pallas as pl

_B = 1024
_D = 64
_C = 16
_K = 25
_MIN_CONF = 0.25
_BIG = 3.0e38


def _overlap_body(enc_ref, cat_ref, ent_ref, ncomp_ref):
    enc = enc_ref[...]
    cat = cat_ref[...]

    sq = jnp.sum(enc * enc, axis=1)
    g = jnp.dot(enc, enc.T, preferred_element_type=jnp.float32)
    d2 = sq[:, None] + sq[None, :] - 2.0 * g

    def next_distinct(_, carry):
        m, thr = carry
        gt = d2 > m[:, None]
        mn = jnp.min(jnp.where(gt, d2, _BIG), axis=1)
        n_above = jnp.sum(gt.astype(jnp.float32), axis=1)
        thr = jnp.where(float(_B) - n_above <= float(_K), mn, thr)
        return mn, thr

    minus_one = jnp.full((_B,), -1.0, jnp.float32)
    _, thresh2 = jax.lax.fori_loop(
        0, _K + 1, next_distinct, (minus_one, minus_one)
    )

    thresh = jnp.sqrt(jnp.maximum(thresh2, 0.0))
    dist = jnp.sqrt(jnp.maximum(d2, 0.0))
    mask = (dist < thresh[:, None]).astype(jnp.float32)
    counts = jnp.sum(mask, axis=1)

    cidx = jax.lax.broadcasted_iota(jnp.int32, (_B, _C), 1)
    maxg = jnp.max(cat, axis=1)
    hard = jnp.min(jnp.where(cat == maxg[:, None], cidx, _C), axis=1)
    onehot = (cidx == hard[:, None]).astype(jnp.float32)

    bins = jnp.dot(mask, onehot, preferred_element_type=jnp.float32)
    bins = bins / counts[:, None]
    ent = -jnp.sum(bins * jnp.log(bins + 1e-5), axis=1)
    ent_ref[...] = ent[:, None]

    conf = (maxg >= _MIN_CONF).astype(jnp.float32)
    populated = jnp.sum(onehot * conf[:, None], axis=0)
    ncomp_ref[...] = jnp.sum((populated > 0.0).astype(jnp.float32)).reshape(1, 1)


def kernel(encodings, categorical):
    ent, ncomp = pl.pallas_call(
        _overlap_body,
        out_shape=[
            jax.ShapeDtypeStruct((_B, 1), jnp.float32),
            jax.ShapeDtypeStruct((1, 1), jnp.float32),
        ],
    )(encodings, categorical)
    return encodings, ent.reshape(_B), ncomp.reshape(())
